# baseline (speedup 1.0000x reference)
import functools

import jax
import jax.numpy as jnp
from jax import lax
from jax.experimental import pallas as pl
from jax.experimental.pallas import tpu as pltpu
from jax.experimental.pallas import tpu_sc as plsc

N = 10000
E = 320000
F = 128
NCLASS = 40

NC = 2
NS = 16
NW = NC * NS
EPT = E // NW
CHUNK = 128
NCHUNK = 80
EPTP = NCHUNK * CHUNK
PAD = EPTP - EPT
NPAD = 10240
RPS = NPAD // NS
NB = 2
QCH = 16
NQ = NCHUNK // QCH

BN = 1000


def _sc_mesh():
    return plsc.VectorSubcoreMesh(
        core_axis_name="c", subcore_axis_name="s", num_cores=NC, num_subcores=NS
    )


@functools.cache
def _make_spmm_sc():
    @functools.partial(
        pl.kernel,
        out_type=jax.ShapeDtypeStruct((NC * NPAD, F), jnp.float32),
        mesh=_sc_mesh(),
        scratch_types=[
            pltpu.VMEM((QCH, CHUNK), jnp.int32),
            pltpu.VMEM((QCH, CHUNK), jnp.int32),
            pltpu.VMEM((NB, CHUNK, F), jnp.float32),
            pltpu.VMEM_SHARED((NPAD, F), jnp.float32),
            pltpu.SemaphoreType.DMA,
            pltpu.SemaphoreType.DMA,
            pltpu.SemaphoreType.DMA,
            pltpu.SemaphoreType.DMA,
        ],
    )
    def _spmm_body(p_hbm, src_hbm, dst_hbm, zeros_hbm, out_hbm,
                   idx_s, idx_d, rows, acc, g0, g1, t0, t1):
        gsems = (g0, g1)
        ssems = (t0, t1)
        cid = lax.axis_index("c")
        sid = lax.axis_index("s")
        wid = cid * NS + sid
        pltpu.sync_copy(zeros_hbm, acc.at[pl.ds(sid * RPS, RPS)])
        plsc.subcore_barrier()

        def quarter(q, carry):
            qoff = pl.multiple_of(q * QCH, 8)
            pltpu.sync_copy(src_hbm.at[wid, pl.ds(qoff, QCH)], idx_s)
            pltpu.sync_copy(dst_hbm.at[wid, pl.ds(qoff, QCH)], idx_d)
            pltpu.async_copy(p_hbm.at[idx_s.at[0]], rows.at[0], gsems[0])

            def group(g, c2):
                for b in range(NB):
                    j = g * NB + b
                    nb = (b + 1) % NB
                    pltpu.make_async_copy(
                        p_hbm.at[idx_s.at[j]], rows.at[b], gsems[b]
                    ).wait()
                    pltpu.async_copy(
                        rows.at[b], acc.at[idx_d.at[j]], ssems[b], add=True
                    )

                    @pl.when(j + 1 < QCH)
                    def _():
                        @pl.when(j >= 1)
                        def _():
                            pltpu.make_async_copy(
                                rows.at[nb], acc.at[idx_d.at[j]], ssems[nb]
                            ).wait()

                        pltpu.async_copy(
                            p_hbm.at[idx_s.at[j + 1]], rows.at[nb], gsems[nb]
                        )
                return c2

            lax.fori_loop(0, QCH // NB, group, 0)
            pltpu.make_async_copy(rows.at[0], acc.at[idx_d.at[0]], ssems[0]).wait()
            pltpu.make_async_copy(rows.at[1], acc.at[idx_d.at[0]], ssems[1]).wait()
            return carry

        lax.fori_loop(0, NQ, quarter, 0)
        plsc.subcore_barrier()
        pltpu.sync_copy(
            acc.at[pl.ds(sid * RPS, RPS)],
            out_hbm.at[pl.ds(cid * NPAD + sid * RPS, RPS)],
        )

    return _spmm_body


def _spmm_sc(p, src, dst, zeros_stripe):
    return _make_spmm_sc()(p, src, dst, zeros_stripe)


@functools.cache
def _make_deg_sc():
    @functools.partial(
        pl.kernel,
        out_type=jax.ShapeDtypeStruct((NC * NPAD, F), jnp.float32),
        mesh=_sc_mesh(),
        scratch_types=[
            pltpu.VMEM((NCHUNK, CHUNK), jnp.int32),
            pltpu.VMEM((CHUNK, F), jnp.float32),
            pltpu.VMEM_SHARED((NPAD, F), jnp.float32),
        ],
    )
    def _deg_body(dst_hbm, zeros_hbm, ones_hbm, out_hbm, idx_d, rows, acc):
        cid = lax.axis_index("c")
        sid = lax.axis_index("s")
        wid = cid * NS + sid
        pltpu.sync_copy(dst_hbm.at[wid], idx_d)
        pltpu.sync_copy(zeros_hbm, acc.at[pl.ds(sid * RPS, RPS)])
        pltpu.sync_copy(ones_hbm, rows)
        plsc.subcore_barrier()

        def body(j, carry):
            pltpu.sync_copy(rows, acc.at[idx_d.at[j]], add=True)
            return carry

        lax.fori_loop(0, NCHUNK, body, 0)
        plsc.subcore_barrier()
        pltpu.sync_copy(
            acc.at[pl.ds(sid * RPS, RPS)],
            out_hbm.at[pl.ds(cid * NPAD + sid * RPS, RPS)],
        )

    return _deg_body


def _deg_sc(dst, zeros_stripe, ones_rows):
    return _make_deg_sc()(dst, zeros_stripe, ones_rows)


def _dinv_body(d0_ref, d1_ref, o_ref):
    deg = d0_ref[...] + d1_ref[...] + 1.0
    o_ref[...] = lax.rsqrt(deg)


def _first_body(x_ref, w_ref, dinv_ref, g_ref, p_ref):
    g = jnp.dot(x_ref[...], w_ref[...], preferred_element_type=jnp.float32)
    g_ref[...] = g
    p_ref[...] = g * dinv_ref[...]


def _mid_body(a0_ref, a1_ref, g_ref, b_ref, dinv_ref, w_ref, go_ref, po_ref):
    dinv = dinv_ref[...]
    h = dinv * (a0_ref[...] + a1_ref[...]) + (dinv * dinv) * g_ref[...] + b_ref[...]
    h = jnp.maximum(h, 0.0)
    g = jnp.dot(h, w_ref[...], preferred_element_type=jnp.float32)
    go_ref[...] = g
    po_ref[...] = g * dinv


def _final_body(a0_ref, a1_ref, g_ref, b_ref, dinv_ref, o_ref):
    dinv = dinv_ref[...]
    z = dinv * (a0_ref[...] + a1_ref[...]) + (dinv * dinv) * g_ref[...] + b_ref[...]
    z = z[:, :NCLASS]
    m = jnp.max(z, axis=1, keepdims=True)
    e = jnp.exp(z - m)
    o_ref[...] = z - m - jnp.log(jnp.sum(e, axis=1, keepdims=True))


def _row_spec(width):
    return pl.BlockSpec((BN, width), lambda i: (i, 0))


def _full_spec(shape):
    return pl.BlockSpec(shape, lambda i: (0, 0))


def _dinv_tc(d0, d1):
    return pl.pallas_call(
        _dinv_body,
        out_shape=jax.ShapeDtypeStruct((N, 1), jnp.float32),
        grid=(N // BN,),
        in_specs=[_row_spec(1), _row_spec(1)],
        out_specs=_row_spec(1),
    )(d0, d1)


def _first_tc(x, w, dinv):
    return pl.pallas_call(
        _first_body,
        out_shape=[
            jax.ShapeDtypeStruct((N, F), jnp.float32),
            jax.ShapeDtypeStruct((N, F), jnp.float32),
        ],
        grid=(N // BN,),
        in_specs=[_row_spec(F), _full_spec((F, F)), _row_spec(1)],
        out_specs=[_row_spec(F), _row_spec(F)],
    )(x, w, dinv)


def _mid_tc(a0, a1, g, b, dinv, w):
    return pl.pallas_call(
        _mid_body,
        out_shape=[
            jax.ShapeDtypeStruct((N, F), jnp.float32),
            jax.ShapeDtypeStruct((N, F), jnp.float32),
        ],
        grid=(N // BN,),
        in_specs=[_row_spec(F), _row_spec(F), _row_spec(F),
                  _full_spec((1, F)), _row_spec(1), _full_spec((F, F))],
        out_specs=[_row_spec(F), _row_spec(F)],
    )(a0, a1, g, b, dinv, w)


def _final_tc(a0, a1, g, b, dinv):
    return pl.pallas_call(
        _final_body,
        out_shape=jax.ShapeDtypeStruct((N, NCLASS), jnp.float32),
        grid=(N // BN,),
        in_specs=[_row_spec(F), _row_spec(F), _row_spec(F),
                  _full_spec((1, F)), _row_spec(1)],
        out_specs=_row_spec(NCLASS),
    )(a0, a1, g, b, dinv)


def kernel(x, edge_index, W1, b1, W2, b2, W3, b3, W4, b4, W5, b5, W6, b6,
           W7, b7, W8, b8):
    ei = edge_index.astype(jnp.int32)
    src = jnp.pad(ei[0].reshape(NW, EPT), ((0, 0), (0, PAD)),
                  constant_values=0).reshape(NW, NCHUNK, CHUNK)
    dst = jnp.pad(ei[1].reshape(NW, EPT), ((0, 0), (0, PAD)),
                  constant_values=N).reshape(NW, NCHUNK, CHUNK)

    zeros_stripe = jnp.zeros((RPS, F), jnp.float32)
    ones_rows = jnp.ones((CHUNK, F), jnp.float32)

    W8p = jnp.zeros((F, F), jnp.float32).at[:, :NCLASS].set(W8)
    b8p = jnp.zeros((1, F), jnp.float32).at[:, :NCLASS].set(b8[None, :])

    Ws = [W1, W2, W3, W4, W5, W6, W7, W8p]
    bs = [b1[None, :], b2[None, :], b3[None, :], b4[None, :], b5[None, :],
          b6[None, :], b7[None, :]]

    degacc = _deg_sc(dst, zeros_stripe, ones_rows)
    dinv = _dinv_tc(degacc[:N, :1], degacc[NPAD:NPAD + N, :1])

    g, p = _first_tc(x, Ws[0], dinv)
    for l in range(1, 8):
        acc = _spmm_sc(p, src, dst, zeros_stripe)
        g, p = _mid_tc(acc[:N], acc[NPAD:NPAD + N], g, bs[l - 1], dinv, Ws[l])
    acc = _spmm_sc(p, src, dst, zeros_stripe)
    return _final_tc(acc[:N], acc[NPAD:NPAD + N], g, b8p, dinv)


# speedup vs baseline: 2.3407x; 2.3407x over previous
import functools

import jax
import jax.numpy as jnp
from jax import lax
from jax.experimental import pallas as pl
from jax.experimental.pallas import tpu as pltpu
from jax.experimental.pallas import tpu_sc as plsc

N = 10000
E = 320000
F = 128
NCLASS = 40

NC = 2
NS = 16
NW = NC * NS
EPT = E // NW
CHUNK = 80
NCHUNK = EPT // CHUNK
NPAD = 10240
RPS = NPAD // NS
NB = 2

BN = 1000


def _sc_mesh():
    return plsc.VectorSubcoreMesh(
        core_axis_name="c", subcore_axis_name="s", num_cores=NC, num_subcores=NS
    )


@functools.cache
def _make_spmm_sc():
    @functools.partial(
        pl.kernel,
        out_type=jax.ShapeDtypeStruct((NC * NPAD, F), jnp.float32),
        mesh=_sc_mesh(),
        scratch_types=[
            pltpu.VMEM((EPT,), jnp.int32),
            pltpu.VMEM((NCHUNK, CHUNK), jnp.int32),
            pltpu.VMEM((NB, CHUNK, F), jnp.float32),
            pltpu.VMEM_SHARED((NPAD, F), jnp.float32),
            pltpu.SemaphoreType.DMA,
            pltpu.SemaphoreType.DMA,
            pltpu.SemaphoreType.DMA,
            pltpu.SemaphoreType.DMA,
        ],
    )
    def _spmm_body(p_hbm, src_hbm, dst_hbm, zeros_hbm, out_hbm,
                   idx_s, idx_d, rows, acc, g0, g1, t0, t1):
        gsems = (g0, g1)
        ssems = (t0, t1)
        cid = lax.axis_index("c")
        sid = lax.axis_index("s")
        wid = cid * NS + sid
        pltpu.sync_copy(src_hbm.at[pl.ds(wid * EPT, EPT)], idx_s)
        pltpu.sync_copy(dst_hbm.at[wid], idx_d)
        pltpu.sync_copy(zeros_hbm, acc.at[pl.ds(sid * RPS, RPS)])
        plsc.subcore_barrier()

        def gather(j, b):
            pltpu.async_copy(
                p_hbm.at[idx_s.at[pl.ds(j * CHUNK, CHUNK)]], rows.at[b],
                gsems[b],
            )

        def chunk_step(j, b):
            b1 = 1 - b
            pltpu.make_async_copy(
                p_hbm.at[idx_s.at[pl.ds(j * CHUNK, CHUNK)]], rows.at[b],
                gsems[b],
            ).wait()
            pltpu.async_copy(rows.at[b], acc.at[idx_d.at[j]], ssems[b],
                             add=True)

            @pl.when(j + 1 < NCHUNK)
            def _():
                @pl.when(j >= 1)
                def _():
                    pltpu.make_async_copy(
                        rows.at[b1], acc.at[idx_d.at[j]], ssems[b1]
                    ).wait()

                gather(j + 1, b1)

        gather(0, 0)

        def group(g, carry):
            chunk_step(g * 2, 0)
            chunk_step(g * 2 + 1, 1)
            return carry

        lax.fori_loop(0, NCHUNK // 2, group, 0)
        chunk_step(NCHUNK - 1, 0)
        pltpu.make_async_copy(rows.at[0], acc.at[idx_d.at[0]], ssems[0]).wait()
        pltpu.make_async_copy(rows.at[1], acc.at[idx_d.at[0]], ssems[1]).wait()
        plsc.subcore_barrier()
        pltpu.sync_copy(
            acc.at[pl.ds(sid * RPS, RPS)],
            out_hbm.at[pl.ds(cid * NPAD + sid * RPS, RPS)],
        )

    return _spmm_body


def _spmm_sc(p, src, dst, zeros_stripe):
    return _make_spmm_sc()(p, src, dst, zeros_stripe)


@functools.cache
def _make_deg_sc():
    @functools.partial(
        pl.kernel,
        out_type=jax.ShapeDtypeStruct((NC * NPAD, F), jnp.float32),
        mesh=_sc_mesh(),
        scratch_types=[
            pltpu.VMEM((NCHUNK, CHUNK), jnp.int32),
            pltpu.VMEM((CHUNK, F), jnp.float32),
            pltpu.VMEM_SHARED((NPAD, F), jnp.float32),
        ],
    )
    def _deg_body(dst_hbm, zeros_hbm, ones_hbm, out_hbm, idx_d, rows, acc):
        cid = lax.axis_index("c")
        sid = lax.axis_index("s")
        wid = cid * NS + sid
        pltpu.sync_copy(dst_hbm.at[wid], idx_d)
        pltpu.sync_copy(zeros_hbm, acc.at[pl.ds(sid * RPS, RPS)])
        pltpu.sync_copy(ones_hbm, rows)
        plsc.subcore_barrier()

        def body(j, carry):
            pltpu.sync_copy(rows, acc.at[idx_d.at[j]], add=True)
            return carry

        lax.fori_loop(0, NCHUNK, body, 0)
        plsc.subcore_barrier()
        pltpu.sync_copy(
            acc.at[pl.ds(sid * RPS, RPS)],
            out_hbm.at[pl.ds(cid * NPAD + sid * RPS, RPS)],
        )

    return _deg_body


def _deg_sc(dst, zeros_stripe, ones_rows):
    return _make_deg_sc()(dst, zeros_stripe, ones_rows)


def _dinv_body(d0_ref, d1_ref, o_ref):
    deg = d0_ref[...] + d1_ref[...] + 1.0
    o_ref[...] = lax.rsqrt(deg)


def _first_body(x_ref, w_ref, dinv_ref, g_ref, p_ref):
    g = jnp.dot(x_ref[...], w_ref[...], preferred_element_type=jnp.float32)
    g_ref[...] = g
    p_ref[...] = g * dinv_ref[...]


def _mid_body(a0_ref, a1_ref, g_ref, b_ref, dinv_ref, w_ref, go_ref, po_ref):
    dinv = dinv_ref[...]
    h = dinv * (a0_ref[...] + a1_ref[...]) + (dinv * dinv) * g_ref[...] + b_ref[...]
    h = jnp.maximum(h, 0.0)
    g = jnp.dot(h, w_ref[...], preferred_element_type=jnp.float32)
    go_ref[...] = g
    po_ref[...] = g * dinv


def _final_body(a0_ref, a1_ref, g_ref, b_ref, dinv_ref, o_ref):
    dinv = dinv_ref[...]
    z = dinv * (a0_ref[...] + a1_ref[...]) + (dinv * dinv) * g_ref[...] + b_ref[...]
    z = z[:, :NCLASS]
    m = jnp.max(z, axis=1, keepdims=True)
    e = jnp.exp(z - m)
    o_ref[...] = z - m - jnp.log(jnp.sum(e, axis=1, keepdims=True))


def _row_spec(width):
    return pl.BlockSpec((BN, width), lambda i: (i, 0))


def _full_spec(shape):
    return pl.BlockSpec(shape, lambda i: (0, 0))


def _dinv_tc(d0, d1):
    return pl.pallas_call(
        _dinv_body,
        out_shape=jax.ShapeDtypeStruct((N, 1), jnp.float32),
        grid=(N // BN,),
        in_specs=[_row_spec(1), _row_spec(1)],
        out_specs=_row_spec(1),
    )(d0, d1)


def _first_tc(x, w, dinv):
    return pl.pallas_call(
        _first_body,
        out_shape=[
            jax.ShapeDtypeStruct((N, F), jnp.float32),
            jax.ShapeDtypeStruct((N, F), jnp.float32),
        ],
        grid=(N // BN,),
        in_specs=[_row_spec(F), _full_spec((F, F)), _row_spec(1)],
        out_specs=[_row_spec(F), _row_spec(F)],
    )(x, w, dinv)


def _mid_tc(a0, a1, g, b, dinv, w):
    return pl.pallas_call(
        _mid_body,
        out_shape=[
            jax.ShapeDtypeStruct((N, F), jnp.float32),
            jax.ShapeDtypeStruct((N, F), jnp.float32),
        ],
        grid=(N // BN,),
        in_specs=[_row_spec(F), _row_spec(F), _row_spec(F),
                  _full_spec((1, F)), _row_spec(1), _full_spec((F, F))],
        out_specs=[_row_spec(F), _row_spec(F)],
    )(a0, a1, g, b, dinv, w)


def _final_tc(a0, a1, g, b, dinv):
    return pl.pallas_call(
        _final_body,
        out_shape=jax.ShapeDtypeStruct((N, NCLASS), jnp.float32),
        grid=(N // BN,),
        in_specs=[_row_spec(F), _row_spec(F), _row_spec(F),
                  _full_spec((1, F)), _row_spec(1)],
        out_specs=_row_spec(NCLASS),
    )(a0, a1, g, b, dinv)


def kernel(x, edge_index, W1, b1, W2, b2, W3, b3, W4, b4, W5, b5, W6, b6,
           W7, b7, W8, b8):
    ei = edge_index.astype(jnp.int32)
    src = ei[0]
    dst = ei[1].reshape(NW, NCHUNK, CHUNK)

    zeros_stripe = jnp.zeros((RPS, F), jnp.float32)
    ones_rows = jnp.ones((CHUNK, F), jnp.float32)

    W8p = jnp.zeros((F, F), jnp.float32).at[:, :NCLASS].set(W8)
    b8p = jnp.zeros((1, F), jnp.float32).at[:, :NCLASS].set(b8[None, :])

    Ws = [W1, W2, W3, W4, W5, W6, W7, W8p]
    bs = [b1[None, :], b2[None, :], b3[None, :], b4[None, :], b5[None, :],
          b6[None, :], b7[None, :]]

    degacc = _deg_sc(dst, zeros_stripe, ones_rows)
    dinv = _dinv_tc(degacc[:N, :1], degacc[NPAD:NPAD + N, :1])

    g, p = _first_tc(x, Ws[0], dinv)
    for l in range(1, 8):
        acc = _spmm_sc(p, src, dst, zeros_stripe)
        g, p = _mid_tc(acc[:N], acc[NPAD:NPAD + N], g, bs[l - 1], dinv, Ws[l])
    acc = _spmm_sc(p, src, dst, zeros_stripe)
    return _final_tc(acc[:N], acc[NPAD:NPAD + N], g, b8p, dinv)


# speedup vs baseline: 2.3569x; 1.0069x over previous
import functools

import jax
import jax.numpy as jnp
from jax import lax
from jax.experimental import pallas as pl
from jax.experimental.pallas import tpu as pltpu
from jax.experimental.pallas import tpu_sc as plsc

N = 10000
E = 320000
F = 128
NCLASS = 40

NC = 2
NS = 16
NW = NC * NS
EPT = E // NW
CHUNK = 80
NCHUNK = EPT // CHUNK
NPAD = 10240
RPS = NPAD // NS
NB = 2

BN = 1000


def _sc_mesh():
    return plsc.VectorSubcoreMesh(
        core_axis_name="c", subcore_axis_name="s", num_cores=NC, num_subcores=NS
    )


@functools.cache
def _make_spmm_sc():
    @functools.partial(
        pl.kernel,
        out_type=jax.ShapeDtypeStruct((NC * NPAD, F), jnp.float32),
        mesh=_sc_mesh(),
        scratch_types=[
            pltpu.VMEM((EPT,), jnp.int32),
            pltpu.VMEM((NCHUNK, CHUNK), jnp.int32),
            pltpu.VMEM((NB, CHUNK, F), jnp.float32),
            pltpu.VMEM_SHARED((NPAD, F), jnp.float32),
            pltpu.SemaphoreType.DMA,
            pltpu.SemaphoreType.DMA,
            pltpu.SemaphoreType.DMA,
            pltpu.SemaphoreType.DMA,
        ],
    )
    def _spmm_body(p_hbm, src_hbm, dst_hbm, zeros_hbm, out_hbm,
                   idx_s, idx_d, rows, acc, g0, g1, t0, t1):
        gsems = (g0, g1)
        ssems = (t0, t1)
        cid = lax.axis_index("c")
        sid = lax.axis_index("s")
        wid = cid * NS + sid
        pltpu.sync_copy(src_hbm.at[pl.ds(wid * EPT, EPT)], idx_s)
        pltpu.sync_copy(dst_hbm.at[wid], idx_d)
        pltpu.sync_copy(zeros_hbm, acc.at[pl.ds(sid * RPS, RPS)])
        plsc.subcore_barrier()

        def gather(j, b):
            pltpu.async_copy(
                p_hbm.at[idx_s.at[pl.ds(j * CHUNK, CHUNK)]], rows.at[b],
                gsems[b],
            )

        def chunk_step(j, b):
            b1 = 1 - b
            pltpu.make_async_copy(
                p_hbm.at[idx_s.at[pl.ds(j * CHUNK, CHUNK)]], rows.at[b],
                gsems[b],
            ).wait()
            pltpu.async_copy(rows.at[b], acc.at[idx_d.at[j]], ssems[b],
                             add=True)

            @pl.when(j + 1 < NCHUNK)
            def _():
                @pl.when(j >= 1)
                def _():
                    pltpu.make_async_copy(
                        rows.at[b1], acc.at[idx_d.at[j]], ssems[b1]
                    ).wait()

                gather(j + 1, b1)

        gather(0, 0)

        def group(g, carry):
            chunk_step(g * 2, 0)
            chunk_step(g * 2 + 1, 1)
            return carry

        lax.fori_loop(0, NCHUNK // 2, group, 0)
        chunk_step(NCHUNK - 1, 0)
        pltpu.make_async_copy(rows.at[0], acc.at[idx_d.at[0]], ssems[0]).wait()
        pltpu.make_async_copy(rows.at[1], acc.at[idx_d.at[0]], ssems[1]).wait()
        plsc.subcore_barrier()
        pltpu.sync_copy(
            acc.at[pl.ds(sid * RPS, RPS)],
            out_hbm.at[pl.ds(cid * NPAD + sid * RPS, RPS)],
        )

    return _spmm_body


def _spmm_sc(p, src, dst, zeros_stripe):
    return _make_spmm_sc()(p, src, dst, zeros_stripe)


@functools.cache
def _make_deg_sc():
    @functools.partial(
        pl.kernel,
        out_type=jax.ShapeDtypeStruct((NC * NPAD, F), jnp.float32),
        mesh=_sc_mesh(),
        scratch_types=[
            pltpu.VMEM((NCHUNK, CHUNK), jnp.int32),
            pltpu.VMEM((CHUNK, F), jnp.float32),
            pltpu.VMEM_SHARED((NPAD, F), jnp.float32),
            pltpu.SemaphoreType.DMA,
            pltpu.SemaphoreType.DMA,
        ],
    )
    def _deg_body(dst_hbm, zeros_hbm, ones_hbm, out_hbm, idx_d, rows, acc,
                  t0, t1):
        ssems = (t0, t1)
        cid = lax.axis_index("c")
        sid = lax.axis_index("s")
        wid = cid * NS + sid
        pltpu.sync_copy(dst_hbm.at[wid], idx_d)
        pltpu.sync_copy(zeros_hbm, acc.at[pl.ds(sid * RPS, RPS)])
        pltpu.sync_copy(ones_hbm, rows)
        plsc.subcore_barrier()

        def fire(j, b):
            pltpu.async_copy(rows, acc.at[idx_d.at[j]], ssems[b], add=True)

        def drain(b):
            pltpu.make_async_copy(rows, acc.at[idx_d.at[0]], ssems[b]).wait()

        fire(0, 0)
        fire(1, 1)

        def body(g, carry):
            for b in range(2):
                j = g * 2 + b
                drain(b)
                fire(j, b)
            return carry

        lax.fori_loop(1, (NCHUNK - 1) // 2, body, 0)
        drain(0)
        fire(NCHUNK - 1, 0)
        drain(1)
        drain(0)
        plsc.subcore_barrier()
        pltpu.sync_copy(
            acc.at[pl.ds(sid * RPS, RPS)],
            out_hbm.at[pl.ds(cid * NPAD + sid * RPS, RPS)],
        )

    return _deg_body


def _deg_sc(dst, zeros_stripe, ones_rows):
    return _make_deg_sc()(dst, zeros_stripe, ones_rows)


def _first_body(x_ref, w_ref, d0_ref, d1_ref, g_ref, p_ref, dinv_ref):
    dinv = lax.rsqrt(d0_ref[...] + d1_ref[...] + 1.0)
    dinv_ref[...] = dinv
    g = jnp.dot(x_ref[...], w_ref[...], preferred_element_type=jnp.float32)
    g_ref[...] = g
    p_ref[...] = g * dinv


def _mid_body(a0_ref, a1_ref, g_ref, b_ref, dinv_ref, w_ref, go_ref, po_ref):
    dinv = dinv_ref[...]
    h = dinv * (a0_ref[...] + a1_ref[...]) + (dinv * dinv) * g_ref[...] + b_ref[...]
    h = jnp.maximum(h, 0.0)
    g = jnp.dot(h, w_ref[...], preferred_element_type=jnp.float32)
    go_ref[...] = g
    po_ref[...] = g * dinv


def _final_body(a0_ref, a1_ref, g_ref, b_ref, dinv_ref, o_ref):
    dinv = dinv_ref[...]
    z = dinv * (a0_ref[...] + a1_ref[...]) + (dinv * dinv) * g_ref[...] + b_ref[...]
    z = z[:, :NCLASS]
    m = jnp.max(z, axis=1, keepdims=True)
    e = jnp.exp(z - m)
    o_ref[...] = z - m - jnp.log(jnp.sum(e, axis=1, keepdims=True))


def _row_spec(width):
    return pl.BlockSpec((BN, width), lambda i: (i, 0))


def _full_spec(shape):
    return pl.BlockSpec(shape, lambda i: (0, 0))


def _first_tc(x, w, d0, d1):
    return pl.pallas_call(
        _first_body,
        out_shape=[
            jax.ShapeDtypeStruct((N, F), jnp.float32),
            jax.ShapeDtypeStruct((N, F), jnp.float32),
            jax.ShapeDtypeStruct((N, 1), jnp.float32),
        ],
        grid=(N // BN,),
        in_specs=[_row_spec(F), _full_spec((F, F)), _row_spec(1), _row_spec(1)],
        out_specs=[_row_spec(F), _row_spec(F), _row_spec(1)],
    )(x, w, d0, d1)


def _mid_tc(a0, a1, g, b, dinv, w):
    return pl.pallas_call(
        _mid_body,
        out_shape=[
            jax.ShapeDtypeStruct((N, F), jnp.float32),
            jax.ShapeDtypeStruct((N, F), jnp.float32),
        ],
        grid=(N // BN,),
        in_specs=[_row_spec(F), _row_spec(F), _row_spec(F),
                  _full_spec((1, F)), _row_spec(1), _full_spec((F, F))],
        out_specs=[_row_spec(F), _row_spec(F)],
    )(a0, a1, g, b, dinv, w)


def _final_tc(a0, a1, g, b, dinv):
    return pl.pallas_call(
        _final_body,
        out_shape=jax.ShapeDtypeStruct((N, NCLASS), jnp.float32),
        grid=(N // BN,),
        in_specs=[_row_spec(F), _row_spec(F), _row_spec(F),
                  _full_spec((1, F)), _row_spec(1)],
        out_specs=_row_spec(NCLASS),
    )(a0, a1, g, b, dinv)


def kernel(x, edge_index, W1, b1, W2, b2, W3, b3, W4, b4, W5, b5, W6, b6,
           W7, b7, W8, b8):
    ei = edge_index.astype(jnp.int32)
    src = ei[0]
    dst = ei[1].reshape(NW, NCHUNK, CHUNK)

    zeros_stripe = jnp.zeros((RPS, F), jnp.float32)
    ones_rows = jnp.ones((CHUNK, F), jnp.float32)

    W8p = jnp.zeros((F, F), jnp.float32).at[:, :NCLASS].set(W8)
    b8p = jnp.zeros((1, F), jnp.float32).at[:, :NCLASS].set(b8[None, :])

    Ws = [W1, W2, W3, W4, W5, W6, W7, W8p]
    bs = [b1[None, :], b2[None, :], b3[None, :], b4[None, :], b5[None, :],
          b6[None, :], b7[None, :]]

    degacc = _deg_sc(dst, zeros_stripe, ones_rows)
    g, p, dinv = _first_tc(x, Ws[0], degacc[:N, :1], degacc[NPAD:NPAD + N, :1])
    for l in range(1, 8):
        acc = _spmm_sc(p, src, dst, zeros_stripe)
        g, p = _mid_tc(acc[:N], acc[NPAD:NPAD + N], g, bs[l - 1], dinv, Ws[l])
    acc = _spmm_sc(p, src, dst, zeros_stripe)
    return _final_tc(acc[:N], acc[NPAD:NPAD + N], g, b8p, dinv)
